# baseline (speedup 1.0000x reference)
import functools

import jax
import jax.numpy as jnp
from jax import lax
from jax.experimental import pallas as pl
from jax.experimental.pallas import tpu as pltpu
from jax.experimental.pallas import tpu_sc as plsc

_MAX_DIS = 512
_DIM = 128
_N = 819200

_NC = 2
_NS = 16
_NW = _NC * _NS
_B_PER_W = _N // _NW
_CHUNK = 320
_NSTEPS = _B_PER_W // _CHUNK
_NBUF = 3
_LANES = 16


def _body(d_hbm, table_hbm, out_hbm, table_sp, idx, rows, sg, so):
    sid = lax.axis_index("s")
    wid = sid * _NC + lax.axis_index("c")
    base = wid * _B_PER_W

    @pl.when(sid == 0)
    def _():
        pltpu.sync_copy(table_hbm, table_sp)

    plsc.subcore_barrier()

    def load_and_clamp(b, g):
        off = base + g * _CHUNK
        pltpu.sync_copy(d_hbm.at[pl.ds(off, _CHUNK)], idx[b])

        def clamp(i, c):
            v = idx[b][pl.ds(i * _LANES, _LANES)]
            idx[b][pl.ds(i * _LANES, _LANES)] = jnp.where(
                v > 1000, _MAX_DIS + 1, jnp.minimum(v, _MAX_DIS)
            )
            return c

        lax.fori_loop(0, _CHUNK // _LANES, clamp, 0)

    def launch_gather(b):
        pltpu.async_copy(table_sp.at[idx[b]], rows[b], sg[b])

    def wait_gather(b):
        pltpu.make_async_copy(table_sp.at[idx[b]], rows[b], sg[b]).wait()

    def start_out(b, g):
        off = base + g * _CHUNK
        pltpu.make_async_copy(
            rows[b], out_hbm.at[pl.ds(off, _CHUNK)], so[b]).start()

    def wait_out(b, g):
        off = base + g * _CHUNK
        pltpu.make_async_copy(
            rows[b], out_hbm.at[pl.ds(off, _CHUNK)], so[b]).wait()

    for g in (0, 1):
        load_and_clamp(g % _NBUF, g)
        launch_gather(g % _NBUF)

    def step(i, carry):
        for k in range(_NBUF):
            gg = _NBUF * i + k
            b = k
            fb = (k + 2) % _NBUF
            wait_gather(b)
            load_and_clamp(fb, gg + 2)
            launch_gather(fb)
            off = base + gg * _CHUNK
            out_cp = pltpu.make_async_copy(
                rows[b], out_hbm.at[pl.ds(off, _CHUNK)], so[b])
            out_cp.start()
            out_cp.wait()
        return carry

    lax.fori_loop(0, (_NSTEPS - 2) // _NBUF, step, 0)

    for g in (_NSTEPS - 2, _NSTEPS - 1):
        b = g % _NBUF
        wait_gather(b)
        start_out(b, g)
        wait_out(b, g)


_mesh = plsc.VectorSubcoreMesh(core_axis_name="c", subcore_axis_name="s")

_gather = functools.partial(
    pl.kernel,
    out_type=jax.ShapeDtypeStruct((_N, _DIM), jnp.float32),
    mesh=_mesh,
    scratch_types=[
        pltpu.VMEM_SHARED((_MAX_DIS + 2, _DIM), jnp.float32),
        [pltpu.VMEM((_CHUNK,), jnp.int32) for _ in range(_NBUF)],
        [pltpu.VMEM((_CHUNK, _DIM), jnp.float32) for _ in range(_NBUF)],
        [pltpu.SemaphoreType.DMA for _ in range(_NBUF)],
        [pltpu.SemaphoreType.DMA for _ in range(_NBUF)],
    ],
)(_body)


def kernel(d, embed_d):
    return _gather(d, embed_d)
